# Initial kernel scaffold; baseline (speedup 1.0000x reference)
import jax
import jax.numpy as jnp
from jax.experimental import pallas as pl


def kernel(features, feature_bank, pose_bank, W1, b1, W2, b2):
    raise NotImplementedError("write your pallas kernel here")


# speedup vs baseline: 19.5127x; 19.5127x over previous
import functools

import jax
import jax.numpy as jnp
from jax.experimental import pallas as pl
from jax.experimental.pallas import tpu as pltpu

_K = 16
_NEG = -1e30


def _body(x_ref, bank_ref, w1a_ref, w1b_ref, b1_ref, w2_ref, b2_ref, o_ref,
          bn_ref, *, nbank, k):
    f32 = jnp.float32
    x = x_ref[:]
    qn = x / jnp.maximum(
        jnp.sqrt(jnp.sum(x * x, axis=1, keepdims=True)), 1e-12)
    bank = bank_ref[:]

    @pl.when(pl.program_id(0) == 0)
    def _():
        bn_ref[:] = bank / jnp.maximum(
            jnp.sqrt(jnp.sum(bank * bank, axis=1, keepdims=True)), 1e-12)

    bn = bn_ref[:]

    sim = jax.lax.dot_general(
        qn, bn, (((1,), (1,)), ((), ())),
        preferred_element_type=f32,
        precision=jax.lax.Precision.DEFAULT)
    col = jax.lax.broadcasted_iota(jnp.int32, sim.shape, 1)
    sim = jnp.where(col < nbank, sim, _NEG)

    cur = sim
    rowmax = None
    m = None
    for i in range(k):
        m = jnp.max(cur, axis=1, keepdims=True)
        if i == 0:
            rowmax = m
        if i < k - 1:
            cur = jnp.where(cur >= m, _NEG, cur)

    mask = sim >= m
    w = jnp.where(mask, jnp.exp(sim - rowmax), 0.0)
    wn = w / jnp.sum(w, axis=1, keepdims=True)

    wnf = jax.lax.dot_general(
        wn, bank, (((1,), (0,)), ((), ())),
        preferred_element_type=f32,
        precision=jax.lax.Precision.DEFAULT)

    h = (jax.lax.dot_general(x, w1a_ref[:], (((1,), (1,)), ((), ())),
                             preferred_element_type=f32,
                             precision=jax.lax.Precision.DEFAULT)
         + jax.lax.dot_general(wnf, w1b_ref[:], (((1,), (1,)), ((), ())),
                               preferred_element_type=f32,
                               precision=jax.lax.Precision.DEFAULT)
         + b1_ref[:])
    h = jnp.maximum(h, 0.0)
    o_ref[:] = (jax.lax.dot_general(h, w2_ref[:], (((1,), (1,)), ((), ())),
                                    preferred_element_type=f32,
                                    precision=jax.lax.Precision.DEFAULT)
                + b2_ref[:])


@jax.jit
def kernel(features, feature_bank, pose_bank, W1, b1, W2, b2):
    del pose_bank
    B, F = features.shape
    nbank = feature_bank.shape[0]
    BANKP = ((nbank + 255) // 256) * 256
    bank_p = jnp.pad(feature_bank, ((0, BANKP - nbank), (0, 0)))

    R = 512
    while B % R:
        R //= 2

    out = pl.pallas_call(
        functools.partial(_body, nbank=nbank, k=_K),
        grid=(B // R,),
        in_specs=[
            pl.BlockSpec((R, F), lambda i: (i, 0)),
            pl.BlockSpec((BANKP, F), lambda i: (0, 0)),
            pl.BlockSpec((F, F), lambda i: (0, 0)),
            pl.BlockSpec((F, F), lambda i: (0, 0)),
            pl.BlockSpec((1, F), lambda i: (0, 0)),
            pl.BlockSpec((F, F), lambda i: (0, 0)),
            pl.BlockSpec((1, F), lambda i: (0, 0)),
        ],
        out_specs=pl.BlockSpec((R, F), lambda i: (i, 0)),
        out_shape=jax.ShapeDtypeStruct((B, F), features.dtype),
        scratch_shapes=[pltpu.VMEM((BANKP, F), jnp.float32)],
    )(features, bank_p, W1[:, :F], W1[:, F:], b1.reshape(1, F),
      W2, b2.reshape(1, F))
    return out


# speedup vs baseline: 22.7770x; 1.1673x over previous
import functools

import jax
import jax.numpy as jnp
from jax.experimental import pallas as pl
from jax.experimental.pallas import tpu as pltpu

_K = 16
_NEG = -1e30


def _body(x_ref, bank_ref, w1a_ref, w1b_ref, b1_ref, w2_ref, b2_ref, o_ref,
          bn_ref, *, nbank, k):
    f32 = jnp.float32
    x = x_ref[:]
    qn = x / jnp.maximum(
        jnp.sqrt(jnp.sum(x * x, axis=1, keepdims=True)), 1e-12)
    bank = bank_ref[:]

    @pl.when(pl.program_id(0) == 0)
    def _():
        bn_ref[:] = bank / jnp.maximum(
            jnp.sqrt(jnp.sum(bank * bank, axis=1, keepdims=True)), 1e-12)

    bn = bn_ref[:]

    sim = jax.lax.dot_general(
        qn, bn, (((1,), (1,)), ((), ())),
        preferred_element_type=f32,
        precision=jax.lax.Precision.DEFAULT)
    col = jax.lax.broadcasted_iota(jnp.int32, sim.shape, 1)
    sim = jnp.where(col < nbank, sim, _NEG)

    nsl = sim.shape[1] // 128
    s = [sim[:, j * 128:(j + 1) * 128] for j in range(nsl)]
    for (a, b) in [(0, 1), (2, 3), (4, 5), (6, 7),
                   (0, 2), (1, 3), (4, 6), (5, 7),
                   (1, 2), (5, 6),
                   (0, 4), (1, 5), (2, 6), (3, 7),
                   (2, 4), (3, 5),
                   (1, 2), (3, 4), (5, 6)]:
        hi = jnp.maximum(s[a], s[b])
        lo = jnp.minimum(s[a], s[b])
        s[a], s[b] = hi, lo

    rowmax = None
    m = None
    for i in range(k):
        m = jnp.max(s[0], axis=1, keepdims=True)
        if i == 0:
            rowmax = m
        if i < k - 1:
            p = s[0] >= m
            for kk in range(nsl - 1):
                s[kk] = jnp.where(p, s[kk + 1], s[kk])
            s[nsl - 1] = jnp.where(p, _NEG, s[nsl - 1])

    mask = sim >= m
    w = jnp.where(mask, jnp.exp(sim - rowmax), 0.0)
    denom = jnp.sum(w, axis=1, keepdims=True)

    wnf = jax.lax.dot_general(
        w, bank, (((1,), (0,)), ((), ())),
        preferred_element_type=f32,
        precision=jax.lax.Precision.DEFAULT) / denom

    h = (jax.lax.dot_general(x, w1a_ref[:], (((1,), (1,)), ((), ())),
                             preferred_element_type=f32,
                             precision=jax.lax.Precision.DEFAULT)
         + jax.lax.dot_general(wnf, w1b_ref[:], (((1,), (1,)), ((), ())),
                               preferred_element_type=f32,
                               precision=jax.lax.Precision.DEFAULT)
         + b1_ref[:])
    h = jnp.maximum(h, 0.0)
    o_ref[:] = (jax.lax.dot_general(h, w2_ref[:], (((1,), (1,)), ((), ())),
                                    preferred_element_type=f32,
                                    precision=jax.lax.Precision.DEFAULT)
                + b2_ref[:])


@jax.jit
def kernel(features, feature_bank, pose_bank, W1, b1, W2, b2):
    del pose_bank
    B, F = features.shape
    nbank = feature_bank.shape[0]
    BANKP = ((nbank + 255) // 256) * 256
    bank_p = jnp.pad(feature_bank, ((0, BANKP - nbank), (0, 0)))

    R = 1024
    while B % R:
        R //= 2

    out = pl.pallas_call(
        functools.partial(_body, nbank=nbank, k=_K),
        grid=(B // R,),
        in_specs=[
            pl.BlockSpec((R, F), lambda i: (i, 0)),
            pl.BlockSpec((BANKP, F), lambda i: (0, 0)),
            pl.BlockSpec((F, F), lambda i: (0, 0)),
            pl.BlockSpec((F, F), lambda i: (0, 0)),
            pl.BlockSpec((1, F), lambda i: (0, 0)),
            pl.BlockSpec((F, F), lambda i: (0, 0)),
            pl.BlockSpec((1, F), lambda i: (0, 0)),
        ],
        out_specs=pl.BlockSpec((R, F), lambda i: (i, 0)),
        out_shape=jax.ShapeDtypeStruct((B, F), features.dtype),
        scratch_shapes=[pltpu.VMEM((BANKP, F), jnp.float32)],
    )(features, bank_p, W1[:, :F], W1[:, F:], b1.reshape(1, F),
      W2, b2.reshape(1, F))
    return out


# speedup vs baseline: 25.2744x; 1.1096x over previous
import functools

import jax
import jax.numpy as jnp
from jax.experimental import pallas as pl
from jax.experimental.pallas import tpu as pltpu

_K = 16
_NEG = -1e30


def _body(x_ref, bank_ref, w1a_ref, w1b_ref, b1_ref, w2_ref, b2_ref, o_ref,
          bn_ref, *, nbank, k):
    f32 = jnp.float32
    x = x_ref[:]
    bank = bank_ref[:]

    @pl.when(pl.program_id(0) == 0)
    def _():
        bn_ref[:] = bank / jnp.maximum(
            jnp.sqrt(jnp.sum(bank * bank, axis=1, keepdims=True)), 1e-12)

    bn = bn_ref[:]

    nch = 8
    ch = x.shape[0] // nch
    sim_parts = []
    for c in range(nch):
        xc = x[c * ch:(c + 1) * ch]
        qc = xc / jnp.maximum(
            jnp.sqrt(jnp.sum(xc * xc, axis=1, keepdims=True)), 1e-12)
        sim_parts.append(jax.lax.dot_general(
            qc, bn, (((1,), (1,)), ((), ())),
            preferred_element_type=f32,
            precision=jax.lax.Precision.DEFAULT))
    sim = jnp.concatenate(sim_parts, axis=0)
    col = jax.lax.broadcasted_iota(jnp.int32, sim.shape, 1)
    sim = jnp.where(col < nbank, sim, _NEG)

    nsl = sim.shape[1] // 128
    s = [sim[:, j * 128:(j + 1) * 128] for j in range(nsl)]
    for (a, b) in [(0, 1), (2, 3), (4, 5), (6, 7),
                   (0, 2), (1, 3), (4, 6), (5, 7),
                   (1, 2), (5, 6),
                   (0, 4), (1, 5), (2, 6), (3, 7),
                   (2, 4), (3, 5),
                   (1, 2), (3, 4), (5, 6)]:
        hi = jnp.maximum(s[a], s[b])
        lo = jnp.minimum(s[a], s[b])
        s[a], s[b] = hi, lo

    rowmax = None
    m = None
    for i in range(k):
        m = jnp.max(s[0], axis=1, keepdims=True)
        if i == 0:
            rowmax = m
        if i < k - 1:
            p = s[0] >= m
            for kk in range(nsl - 1):
                s[kk] = jnp.where(p, s[kk + 1], s[kk])
            s[nsl - 1] = jnp.where(p, _NEG, s[nsl - 1])

    mask = sim >= m
    w = jnp.where(mask, jnp.exp(sim - rowmax), 0.0)
    denom = jnp.sum(w, axis=1, keepdims=True)

    wnf = jax.lax.dot_general(
        w, bank, (((1,), (0,)), ((), ())),
        preferred_element_type=f32,
        precision=jax.lax.Precision.DEFAULT) / denom

    h = (jax.lax.dot_general(x, w1a_ref[:], (((1,), (1,)), ((), ())),
                             preferred_element_type=f32,
                             precision=jax.lax.Precision.DEFAULT)
         + jax.lax.dot_general(wnf, w1b_ref[:], (((1,), (1,)), ((), ())),
                               preferred_element_type=f32,
                               precision=jax.lax.Precision.DEFAULT)
         + b1_ref[:])
    h = jnp.maximum(h, 0.0)
    o_ref[:] = (jax.lax.dot_general(h, w2_ref[:], (((1,), (1,)), ((), ())),
                                    preferred_element_type=f32,
                                    precision=jax.lax.Precision.DEFAULT)
                + b2_ref[:])


@jax.jit
def kernel(features, feature_bank, pose_bank, W1, b1, W2, b2):
    del pose_bank
    B, F = features.shape
    nbank = feature_bank.shape[0]
    BANKP = ((nbank + 255) // 256) * 256
    bank_p = jnp.pad(feature_bank, ((0, BANKP - nbank), (0, 0)))

    R = 1024
    while B % R:
        R //= 2

    out = pl.pallas_call(
        functools.partial(_body, nbank=nbank, k=_K),
        grid=(B // R,),
        in_specs=[
            pl.BlockSpec((R, F), lambda i: (i, 0)),
            pl.BlockSpec((BANKP, F), lambda i: (0, 0)),
            pl.BlockSpec((F, F), lambda i: (0, 0)),
            pl.BlockSpec((F, F), lambda i: (0, 0)),
            pl.BlockSpec((1, F), lambda i: (0, 0)),
            pl.BlockSpec((F, F), lambda i: (0, 0)),
            pl.BlockSpec((1, F), lambda i: (0, 0)),
        ],
        out_specs=pl.BlockSpec((R, F), lambda i: (i, 0)),
        out_shape=jax.ShapeDtypeStruct((B, F), features.dtype),
        scratch_shapes=[pltpu.VMEM((BANKP, F), jnp.float32)],
    )(features, bank_p, W1[:, :F], W1[:, F:], b1.reshape(1, F),
      W2, b2.reshape(1, F))
    return out


# speedup vs baseline: 25.7336x; 1.0182x over previous
import functools

import jax
import jax.numpy as jnp
from jax.experimental import pallas as pl
from jax.experimental.pallas import tpu as pltpu

_K = 16
_NEG = -1e30


def _body(x_ref, bank_ref, w1a_ref, w1b_ref, b1_ref, w2_ref, b2_ref, o_ref,
          bn_ref, *, nbank, k):
    f32 = jnp.float32
    x = x_ref[:]
    bank = bank_ref[:]

    @pl.when(pl.program_id(0) == 0)
    def _():
        bn_ref[:] = bank / jnp.maximum(
            jnp.sqrt(jnp.sum(bank * bank, axis=1, keepdims=True)), 1e-12)

    bn = bn_ref[:]

    nch = 8
    ch = x.shape[0] // nch
    sim_parts = []
    for c in range(nch):
        xc = x[c * ch:(c + 1) * ch]
        qc = xc / jnp.maximum(
            jnp.sqrt(jnp.sum(xc * xc, axis=1, keepdims=True)), 1e-12)
        sim_parts.append(jax.lax.dot_general(
            qc, bn, (((1,), (1,)), ((), ())),
            preferred_element_type=f32,
            precision=jax.lax.Precision.DEFAULT))
    sim = jnp.concatenate(sim_parts, axis=0)
    col = jax.lax.broadcasted_iota(jnp.int32, sim.shape, 1)
    sim = jnp.where(col < nbank, sim, _NEG)

    nsl = sim.shape[1] // 128
    s = [sim[:, j * 128:(j + 1) * 128] for j in range(nsl)]
    for (a, b) in [(0, 1), (2, 3), (4, 5), (6, 7),
                   (0, 2), (1, 3), (4, 6), (5, 7),
                   (1, 2), (5, 6),
                   (0, 4), (1, 5), (2, 6), (3, 7),
                   (2, 4), (3, 5),
                   (1, 2), (3, 4), (5, 6)]:
        hi = jnp.maximum(s[a], s[b])
        lo = jnp.minimum(s[a], s[b])
        s[a], s[b] = hi, lo

    rowmax = None
    m = None
    for i in range(k):
        m = jnp.max(s[0], axis=1, keepdims=True)
        if i == 0:
            rowmax = m
        if i < k - 1:
            p = s[0] >= m
            depth = min(nsl, k - 1 - i)
            for kk in range(depth):
                nxt = s[kk + 1] if kk + 1 < nsl else _NEG
                s[kk] = jnp.where(p, nxt, s[kk])

    mask = sim >= m
    w = jnp.where(mask, jnp.exp(sim - rowmax), 0.0)
    denom = jnp.sum(w, axis=1, keepdims=True)

    wnf = jax.lax.dot_general(
        w, bank, (((1,), (0,)), ((), ())),
        preferred_element_type=f32,
        precision=jax.lax.Precision.DEFAULT) / denom

    h = (jax.lax.dot_general(x, w1a_ref[:], (((1,), (1,)), ((), ())),
                             preferred_element_type=f32,
                             precision=jax.lax.Precision.DEFAULT)
         + jax.lax.dot_general(wnf, w1b_ref[:], (((1,), (1,)), ((), ())),
                               preferred_element_type=f32,
                               precision=jax.lax.Precision.DEFAULT)
         + b1_ref[:])
    h = jnp.maximum(h, 0.0)
    o_ref[:] = (jax.lax.dot_general(h, w2_ref[:], (((1,), (1,)), ((), ())),
                                    preferred_element_type=f32,
                                    precision=jax.lax.Precision.DEFAULT)
                + b2_ref[:])


@jax.jit
def kernel(features, feature_bank, pose_bank, W1, b1, W2, b2):
    del pose_bank
    B, F = features.shape
    nbank = feature_bank.shape[0]
    BANKP = ((nbank + 255) // 256) * 256
    bank_p = jnp.pad(feature_bank, ((0, BANKP - nbank), (0, 0)))

    R = 2048
    while B % R:
        R //= 2

    out = pl.pallas_call(
        functools.partial(_body, nbank=nbank, k=_K),
        grid=(B // R,),
        in_specs=[
            pl.BlockSpec((R, F), lambda i: (i, 0)),
            pl.BlockSpec((BANKP, F), lambda i: (0, 0)),
            pl.BlockSpec((F, F), lambda i: (0, 0)),
            pl.BlockSpec((F, F), lambda i: (0, 0)),
            pl.BlockSpec((1, F), lambda i: (0, 0)),
            pl.BlockSpec((F, F), lambda i: (0, 0)),
            pl.BlockSpec((1, F), lambda i: (0, 0)),
        ],
        out_specs=pl.BlockSpec((R, F), lambda i: (i, 0)),
        out_shape=jax.ShapeDtypeStruct((B, F), features.dtype),
        scratch_shapes=[pltpu.VMEM((BANKP, F), jnp.float32)],
    )(features, bank_p, W1[:, :F], W1[:, F:], b1.reshape(1, F),
      W2, b2.reshape(1, F))
    return out
